# speedup vs baseline: 3.6358x; 3.6358x over previous
import functools

import jax
import jax.numpy as jnp
from jax import lax
from jax.experimental import pallas as pl
from jax.experimental.pallas import tpu as pltpu
from jax.experimental.pallas import tpu_sc as plsc

D3 = 128
BATCH = 16384
NC = 2
NS = 16
NW = NC * NS
BPW = BATCH // NW
CH = 128
NCH = BPW // CH

_mesh = plsc.VectorSubcoreMesh(core_axis_name="c", subcore_axis_name="s")


@functools.partial(
    pl.kernel,
    mesh=_mesh,
    out_type=jax.ShapeDtypeStruct((BATCH, 3 * D3), jnp.float32),
    scratch_types=[
        pltpu.VMEM((NCH, CH), jnp.int32),
        pltpu.VMEM((NCH, CH), jnp.int32),
        pltpu.VMEM((NCH, CH), jnp.int32),
        pltpu.VMEM((CH, D3), jnp.float32),
        pltpu.VMEM((CH, D3), jnp.float32),
        pltpu.VMEM((CH, D3), jnp.float32),
        pltpu.SemaphoreType.DMA,
    ],
)
def _pe3d(xh, yh, zh, xt, yt, zt, out, xi, yi, zi, rx, ry, rz, sem):
    wid = lax.axis_index("s") * NC + lax.axis_index("c")
    base = wid * BPW
    for ci in range(NCH):
        r0 = base + ci * CH
        pltpu.sync_copy(xh.at[pl.ds(r0, CH)], xi.at[ci])
        pltpu.sync_copy(yh.at[pl.ds(r0, CH)], yi.at[ci])
        pltpu.sync_copy(zh.at[pl.ds(r0, CH)], zi.at[ci])
        cx = pltpu.async_copy(xt.at[xi.at[ci]], rx, sem)
        cy = pltpu.async_copy(yt.at[yi.at[ci]], ry, sem)
        cz = pltpu.async_copy(zt.at[zi.at[ci]], rz, sem)
        cx.wait()
        cy.wait()
        cz.wait()
        pltpu.sync_copy(rx, out.at[pl.ds(r0, CH), pl.ds(0, D3)])
        pltpu.sync_copy(ry, out.at[pl.ds(r0, CH), pl.ds(D3, D3)])
        pltpu.sync_copy(rz, out.at[pl.ds(r0, CH), pl.ds(2 * D3, D3)])


def kernel(x, y, z, x_pos, y_pos, z_pos):
    return _pe3d(
        x.astype(jnp.int32),
        y.astype(jnp.int32),
        z.astype(jnp.int32),
        x_pos,
        y_pos,
        z_pos,
    )


# speedup vs baseline: 4.0035x; 1.1011x over previous
import functools

import jax
import jax.numpy as jnp
from jax import lax
from jax.experimental import pallas as pl
from jax.experimental.pallas import tpu as pltpu
from jax.experimental.pallas import tpu_sc as plsc

D3 = 128
BATCH = 16384
NC = 2
NS = 16
NW = NC * NS
BPW = BATCH // NW
CH = 128
NCH = BPW // CH
NBUF = 2

_mesh = plsc.VectorSubcoreMesh(core_axis_name="c", subcore_axis_name="s")


@functools.partial(
    pl.kernel,
    mesh=_mesh,
    out_type=jax.ShapeDtypeStruct((BATCH, 3 * D3), jnp.float32),
    scratch_types=[
        pltpu.VMEM((BPW,), jnp.int32),
        pltpu.VMEM((BPW,), jnp.int32),
        pltpu.VMEM((BPW,), jnp.int32),
        pltpu.VMEM((NBUF, CH, D3), jnp.float32),
        pltpu.VMEM((NBUF, CH, D3), jnp.float32),
        pltpu.VMEM((NBUF, CH, D3), jnp.float32),
        pltpu.SemaphoreType.DMA,
        pltpu.SemaphoreType.DMA,
    ],
)
def _pe3d(xh, yh, zh, xt, yt, zt, out, xi, yi, zi, rx, ry, rz, gsem, wsem):
    wid = lax.axis_index("s") * NC + lax.axis_index("c")
    base = wid * BPW
    pltpu.sync_copy(xh.at[pl.ds(base, BPW)], xi)
    pltpu.sync_copy(yh.at[pl.ds(base, BPW)], yi)
    pltpu.sync_copy(zh.at[pl.ds(base, BPW)], zi)
    writes = [None] * NCH
    for ci in range(NCH):
        b = ci % NBUF
        if ci >= NBUF:
            for w in writes[ci - NBUF]:
                w.wait()
        sl = pl.ds(ci * CH, CH)
        cx = pltpu.async_copy(xt.at[xi.at[sl]], rx.at[b], gsem)
        cy = pltpu.async_copy(yt.at[yi.at[sl]], ry.at[b], gsem)
        cz = pltpu.async_copy(zt.at[zi.at[sl]], rz.at[b], gsem)
        cx.wait()
        cy.wait()
        cz.wait()
        r0 = base + ci * CH
        writes[ci] = (
            pltpu.async_copy(rx.at[b], out.at[pl.ds(r0, CH), pl.ds(0, D3)], wsem),
            pltpu.async_copy(ry.at[b], out.at[pl.ds(r0, CH), pl.ds(D3, D3)], wsem),
            pltpu.async_copy(rz.at[b], out.at[pl.ds(r0, CH), pl.ds(2 * D3, D3)], wsem),
        )
    for ci in range(NCH - NBUF, NCH):
        for w in writes[ci]:
            w.wait()


def kernel(x, y, z, x_pos, y_pos, z_pos):
    return _pe3d(
        x.astype(jnp.int32),
        y.astype(jnp.int32),
        z.astype(jnp.int32),
        x_pos,
        y_pos,
        z_pos,
    )
